# baseline (speedup 1.0000x reference)
import functools

import jax
import jax.numpy as jnp
from jax import lax
from jax.experimental import pallas as pl
from jax.experimental.pallas import tpu as pltpu
from jax.experimental.pallas import tpu_sc as plsc

B = 4096
S = 200
DIM = 128
N = B * S
NPP = 3 * S

_info = plsc.get_sparse_core_info()
NC, NS, L = _info.num_cores, _info.num_subcores, _info.num_lanes
NW = NC * NS
ROWS_PER_W = N // NW
CH = 1024
G = CH // 128
N_CHUNKS = ROWS_PER_W // CH
IDROWS_PER_CH = CH // 128


def _build_pospad_tc(padding_embedding, pos_embedding):
    def body(pad_ref, pos_ref, out_ref):
        out_ref[...] = pad_ref[...][:, None, :] + pos_ref[0:S][None, :, :]

    return pl.pallas_call(
        body,
        out_shape=jax.ShapeDtypeStruct((3, S, DIM), jnp.float32),
    )(padding_embedding, pos_embedding)


def _sc_lookup(embedding, pospad, char2d, pad2d):
    mesh = plsc.VectorSubcoreMesh(core_axis_name="c", subcore_axis_name="s")

    @functools.partial(
        pl.kernel,
        mesh=mesh,
        out_type=jax.ShapeDtypeStruct((N, DIM), jnp.float32),
        scratch_types=[
            pltpu.VMEM((IDROWS_PER_CH, 128), jnp.int32),
            pltpu.VMEM((IDROWS_PER_CH, 128), jnp.int32),
            pltpu.VMEM((IDROWS_PER_CH, 128), jnp.int32),
            pltpu.VMEM((2, 128, DIM), jnp.float32),
            pltpu.VMEM((NPP, DIM), jnp.float32),
            pltpu.SemaphoreType.DMA,
            pltpu.SemaphoreType.DMA,
        ],
    )
    def k(emb_hbm, pp_hbm, char_hbm, pad_hbm, out_hbm,
          char_v, pad_v, ppidx_v, bufa, pp_l, sem_g, sem_o):
        wid = lax.axis_index("s") * NC + lax.axis_index("c")
        w_row0 = wid * ROWS_PER_W

        pltpu.sync_copy(pp_hbm, pp_l)

        def chunk_body(t, carry):
            row0 = pl.multiple_of(w_row0 + t * CH, CH)
            idrow0 = pl.multiple_of(row0 // 128, IDROWS_PER_CH)

            pltpu.sync_copy(char_hbm.at[pl.ds(idrow0, IDROWS_PER_CH)], char_v)
            pltpu.sync_copy(pad_hbm.at[pl.ds(idrow0, IDROWS_PER_CH)], pad_v)

            lane = lax.iota(jnp.int32, L)

            def idx_body(j, _):
                r = j // (128 // L)
                c = j % (128 // L)
                cs = pl.ds(c * L, L)
                base = (row0 + j * L).astype(jnp.int32)
                ppidx_v[r, cs] = pad_v[r, cs] * S + (base + lane) % S
                return _
            lax.fori_loop(0, CH // L, idx_body, 0)

            pltpu.async_copy(emb_hbm.at[char_v.at[0]], bufa.at[0], sem_g)

            def group_body(g, _):
                sp = g % 2

                @pl.when(jnp.logical_and(g >= 1, g < G - 1))
                def _wait_out():
                    pltpu.make_async_copy(
                        bufa.at[1 - sp],
                        out_hbm.at[pl.ds(row0, 128)], sem_o).wait()

                pltpu.make_async_copy(emb_hbm.at[char_v.at[0]],
                                      bufa.at[sp], sem_g).wait()

                @pl.when(g < G - 1)
                def _next_gather():
                    pltpu.async_copy(emb_hbm.at[char_v.at[g + 1]],
                                     bufa.at[1 - sp], sem_g)

                def add_rows(j, _):
                    pv = ppidx_v[g, pl.ds(j * L, L)]
                    for k in range(L):
                        r = j * L + k
                        pprow = pv[k]
                        for c in range(DIM // L):
                            cs = pl.ds(c * L, L)
                            bufa[sp, r, cs] = (bufa[sp, r, cs]
                                               + pp_l[pprow, cs])
                    return _
                lax.fori_loop(0, 128 // L, add_rows, 0)

                pltpu.async_copy(
                    bufa.at[sp], out_hbm.at[pl.ds(row0 + g * 128, 128)],
                    sem_o)
                return _
            lax.fori_loop(0, G, group_body, 0)

            for _d in range(2):
                pltpu.make_async_copy(
                    bufa.at[0], out_hbm.at[pl.ds(row0, 128)], sem_o).wait()
            return carry

        lax.fori_loop(0, N_CHUNKS, chunk_body, 0)

    return k(embedding, pospad, char2d, pad2d)


def kernel(char_ids, pad_ids, embedding, pos_embedding, padding_embedding):
    pospad = _build_pospad_tc(padding_embedding, pos_embedding)
    pospad = pospad.reshape(NPP, DIM)
    char2d = char_ids.reshape(N // 128, 128).astype(jnp.int32)
    pad2d = pad_ids.reshape(N // 128, 128).astype(jnp.int32)
    out = _sc_lookup(embedding, pospad, char2d, pad2d)
    return out.reshape(B, S, DIM)
